# baseline (device time: 9737 ns/iter reference)
import jax
import jax.numpy as jnp
from jax import lax
from jax.experimental import pallas as pl
from jax.experimental.pallas import tpu as pltpu

N_DEV = 4
K = 8
LANES = 128

_BATCHER8 = [
    (0, 1), (2, 3), (4, 5), (6, 7),
    (0, 2), (1, 3), (4, 6), (5, 7),
    (1, 2), (5, 6),
    (0, 4), (1, 5), (2, 6), (3, 7),
    (2, 4), (3, 5),
    (1, 2), (3, 4), (5, 6),
]

_NEG = float("-inf")
_BIG = 30000.0


def _extract_topk(vals, k, store):
    m, n = vals.shape
    col = lax.broadcasted_iota(jnp.int32, (m, n), 1).astype(jnp.bfloat16)
    for t in range(k):
        mx = jnp.max(vals, axis=1, keepdims=True)
        store(t, mx)
        if t < k - 1:
            first = jnp.min(
                jnp.where(vals == mx, col, _BIG), axis=1, keepdims=True
            )
            vals = jnp.where(col == first, _NEG, vals)


def _local_topk_slab(xb, k, store):
    m = xb.shape[0]
    slabs = [xb[:, g * LANES:(g + 1) * LANES] for g in range(8)]
    for i, j in _BATCHER8:
        hi = jnp.maximum(slabs[i], slabs[j])
        lo = jnp.minimum(slabs[i], slabs[j])
        slabs[i], slabs[j] = hi, lo

    col = lax.broadcasted_iota(jnp.int32, (m, LANES), 1).astype(jnp.bfloat16)
    for t in range(k):
        mx = jnp.max(slabs[0], axis=1, keepdims=True)
        store(t, mx)
        if t < k - 1:
            first = jnp.min(
                jnp.where(slabs[0] == mx, col, _BIG), axis=1, keepdims=True
            )
            hit = col == first
            for j in range(7):
                slabs[j] = jnp.where(hit, slabs[j + 1], slabs[j])
            slabs[7] = jnp.where(hit, _NEG, slabs[7])


def kernel(x):
    m, n = x.shape

    def body(x_ref, out_ref, cand_ref, send_sems, recv_sems):
        my = lax.axis_index("i")

        barrier = pltpu.get_barrier_semaphore()
        for p in range(1, N_DEV):
            pl.semaphore_signal(
                barrier,
                inc=1,
                device_id=((my + p) % N_DEV,),
                device_id_type=pl.DeviceIdType.MESH,
            )

        xb = x_ref[:, :].astype(jnp.bfloat16)

        def store_local(t, mxcol):
            cand_ref[0, :, t] = mxcol[:, 0]

        _local_topk_slab(xb, K, store_local)

        pl.semaphore_wait(barrier, N_DEV - 1)

        rdmas = []
        for p in range(1, N_DEV):
            rdma = pltpu.make_async_remote_copy(
                src_ref=cand_ref.at[0],
                dst_ref=cand_ref.at[N_DEV - p],
                send_sem=send_sems.at[p - 1],
                recv_sem=recv_sems.at[p - 1],
                device_id=((my + p) % N_DEV,),
                device_id_type=pl.DeviceIdType.MESH,
            )
            rdma.start()
            rdmas.append(rdma)
        for rdma in rdmas:
            rdma.wait_recv()

        def store_out(t, mxcol):
            out_ref[:, t] = mxcol[:, 0]

        allc = jnp.concatenate(
            [cand_ref[i, :, :] for i in range(N_DEV)], axis=1
        )
        _extract_topk(allc, K, store_out)

        for rdma in rdmas:
            rdma.wait_send()

    return pl.pallas_call(
        body,
        out_shape=jax.ShapeDtypeStruct((m, K), jnp.bfloat16),
        in_specs=[pl.BlockSpec(memory_space=pltpu.VMEM)],
        out_specs=pl.BlockSpec(memory_space=pltpu.VMEM),
        scratch_shapes=[
            pltpu.VMEM((N_DEV, m, K), jnp.bfloat16),
            pltpu.SemaphoreType.DMA((N_DEV - 1,)),
            pltpu.SemaphoreType.DMA((N_DEV - 1,)),
        ],
        compiler_params=pltpu.CompilerParams(collective_id=0),
    )(x)
